# Initial kernel scaffold; baseline (speedup 1.0000x reference)
import jax
import jax.numpy as jnp
from jax.experimental import pallas as pl


def kernel(pos, edge_index):
    raise NotImplementedError("write your pallas kernel here")


# speedup vs baseline: 163.1082x; 163.1082x over previous
import functools

import jax
import jax.numpy as jnp
from jax import lax
from jax.experimental import pallas as pl
from jax.experimental.pallas import tpu as pltpu
from jax.experimental.pallas import tpu_sc as plsc

N_NODES = 100000
N_EDGES = 6400000
NC = 2
NS = 16
NW = NC * NS
L = 16
CB = 128
KB = 8
CHUNK = CB * KB
N_SC = N_EDGES // CHUNK
G_FULL = N_SC // NW
G_REM = N_SC % NW


def _sc_body(pos_hbm, ji_hbm, dvec_hbm, dist_hbm, hist_hbm,
             idx_j, idx_i, rows_j, rows_i, dvec_b, dist_b, hist_v, sem_g):
    cid = lax.axis_index("c")
    sid = lax.axis_index("s")
    wid = cid * NS + sid

    iota = lax.iota(jnp.int32, L)
    zero16 = jnp.zeros((L,), jnp.int32)
    ones16 = jnp.ones((L,), jnp.int32)
    c0 = jnp.zeros((L,), jnp.int32)
    c1 = jnp.full((L,), 1, jnp.int32)
    c2 = jnp.full((L,), 2, jnp.int32)

    def _zero(t, carry):
        hist_v[pl.ds(t * L, L)] = zero16
        return carry

    lax.fori_loop(jnp.int32(0), jnp.int32(N_NODES // L), _zero, 0)

    n_g = jnp.where(wid < G_REM, jnp.int32(G_FULL + 1), jnp.int32(G_FULL))

    def _iter(g, carry):
        sc = g * NW + wid
        pltpu.sync_copy(ji_hbm.at[jnp.int32(0), sc], idx_j)
        pltpu.sync_copy(ji_hbm.at[jnp.int32(1), sc], idx_i)
        for k in range(KB):
            k32 = jnp.int32(k)
            pltpu.async_copy(pos_hbm.at[idx_j.at[k32]], rows_j.at[k32], sem_g)
        for k in range(KB):
            k32 = jnp.int32(k)
            pltpu.async_copy(pos_hbm.at[idx_i.at[k32]], rows_i.at[k32], sem_g)
        for k in range(KB):
            k32 = jnp.int32(k)
            pltpu.make_async_copy(pos_hbm.at[idx_j.at[k32]], rows_j.at[k32],
                                  sem_g).wait()
        for k in range(KB):
            k32 = jnp.int32(k)
            pltpu.make_async_copy(pos_hbm.at[idx_i.at[k32]], rows_i.at[k32],
                                  sem_g).wait()
        for k in range(KB):
            kv = jnp.full((L,), k, jnp.int32)
            for t in range(CB // L):
                rv = iota + (t * L)
                xj = plsc.load_gather(rows_j, [kv, rv, c0])
                yj = plsc.load_gather(rows_j, [kv, rv, c1])
                zj = plsc.load_gather(rows_j, [kv, rv, c2])
                xi = plsc.load_gather(rows_i, [kv, rv, c0])
                yi = plsc.load_gather(rows_i, [kv, rv, c1])
                zi = plsc.load_gather(rows_i, [kv, rv, c2])
                dx = xj - xi
                dy = yj - yi
                dz = zj - zi
                d2 = dx * dx + dy * dy + dz * dz
                bits = lax.bitcast_convert_type(d2, jnp.int32)
                bits = 0x5F3759DF - (bits >> 1)
                y = lax.bitcast_convert_type(bits, jnp.float32)
                y = y * (1.5 - 0.5 * d2 * y * y)
                y = y * (1.5 - 0.5 * d2 * y * y)
                y = y * (1.5 - 0.5 * d2 * y * y)
                dist = jnp.where(d2 > 0.0, d2 * y, 0.0)
                dist_b[jnp.int32(k), pl.ds(t * L, L)] = dist
                plsc.store_scatter(dvec_b, [kv, rv, c0], dx)
                plsc.store_scatter(dvec_b, [kv, rv, c1], dy)
                plsc.store_scatter(dvec_b, [kv, rv, c2], dz)
                ii = idx_i[jnp.int32(k), pl.ds(t * L, L)]
                plsc.addupdate_scatter(hist_v, [ii], ones16)
        pltpu.sync_copy(dvec_b, dvec_hbm.at[sc])
        pltpu.sync_copy(dist_b, dist_hbm.at[sc])
        return carry

    lax.fori_loop(jnp.int32(0), n_g, _iter, 0)
    pltpu.sync_copy(hist_v, hist_hbm.at[wid])


_sc_call = functools.partial(
    pl.kernel,
    out_type=(
        jax.ShapeDtypeStruct((N_SC, KB, CB, 3), jnp.float32),
        jax.ShapeDtypeStruct((N_SC, KB, CB), jnp.float32),
        jax.ShapeDtypeStruct((NW, N_NODES), jnp.int32),
    ),
    mesh=plsc.VectorSubcoreMesh(core_axis_name="c", subcore_axis_name="s",
                                num_cores=NC, num_subcores=NS),
    compiler_params=pltpu.CompilerParams(needs_layout_passes=False,
                                         use_tc_tiling_on_sc=False),
    scratch_types=[
        pltpu.MemorySpace.VMEM((KB, CB), jnp.int32),
        pltpu.MemorySpace.VMEM((KB, CB), jnp.int32),
        pltpu.MemorySpace.VMEM((KB, CB, 8), jnp.float32),
        pltpu.MemorySpace.VMEM((KB, CB, 8), jnp.float32),
        pltpu.MemorySpace.VMEM((KB, CB, 3), jnp.float32),
        pltpu.MemorySpace.VMEM((KB, CB), jnp.float32),
        pltpu.MemorySpace.VMEM((N_NODES,), jnp.int32),
        pltpu.SemaphoreType.DMA,
    ],
)(_sc_body)


def _hist_reduce_body(parts_ref, out_ref):
    out_ref[...] = jnp.sum(parts_ref[...], axis=0, dtype=jnp.int32)


def _hist_reduce(parts):
    return pl.pallas_call(
        _hist_reduce_body,
        out_shape=jax.ShapeDtypeStruct((N_NODES,), jnp.int32),
    )(parts)


def kernel(pos, edge_index):
    ji32 = edge_index.astype(jnp.int32).reshape(2, N_SC, KB, CB)
    pos8 = jnp.pad(pos, ((0, 0), (0, 5)))
    dvec, dist, hist_parts = _sc_call(pos8, ji32)
    neighbors = _hist_reduce(hist_parts).astype(edge_index.dtype)
    zeros = jnp.zeros((N_EDGES, 3), pos.dtype)
    return (edge_index,
            dist.reshape(N_EDGES),
            dvec.reshape(N_EDGES, 3),
            zeros,
            zeros,
            neighbors)


# speedup vs baseline: 269.1423x; 1.6501x over previous
import functools

import jax
import jax.numpy as jnp
from jax import lax
from jax.experimental import pallas as pl
from jax.experimental.pallas import tpu as pltpu
from jax.experimental.pallas import tpu_sc as plsc

N_NODES = 100000
N_EDGES = 6400000
NC = 2
NS = 16
NW = NC * NS
L = 16
CB = 128
KB = 8
CHUNK = CB * KB
N_SC = N_EDGES // CHUNK
G_FULL = N_SC // NW
G_REM = N_SC % NW


def _sc_body(pos_hbm, ji_hbm, dvec_hbm, dist_hbm, hist_hbm,
             idx_j, idx_i, rows_j, rows_i, dxb, dyb, dzb, dist_b, hist_v,
             sem_g):
    cid = lax.axis_index("c")
    sid = lax.axis_index("s")
    wid = cid * NS + sid

    iota = lax.iota(jnp.int32, L)
    zero16 = jnp.zeros((L,), jnp.int32)
    ones16 = jnp.ones((L,), jnp.int32)
    c0 = jnp.zeros((L,), jnp.int32)
    c1 = jnp.full((L,), 1, jnp.int32)
    c2 = jnp.full((L,), 2, jnp.int32)

    def _zero(t, carry):
        hist_v[pl.ds(t * L, L)] = zero16
        return carry

    lax.fori_loop(jnp.int32(0), jnp.int32(N_NODES // L), _zero, 0)

    n_g = jnp.where(wid < G_REM, jnp.int32(G_FULL + 1), jnp.int32(G_FULL))

    def _iter(g, carry):
        sc = g * NW + wid
        pltpu.sync_copy(ji_hbm.at[jnp.int32(0), sc], idx_j)
        pltpu.sync_copy(ji_hbm.at[jnp.int32(1), sc], idx_i)
        for k in range(KB):
            k32 = jnp.int32(k)
            pltpu.async_copy(pos_hbm.at[idx_j.at[k32]], rows_j.at[k32], sem_g)
        for k in range(KB):
            k32 = jnp.int32(k)
            pltpu.async_copy(pos_hbm.at[idx_i.at[k32]], rows_i.at[k32], sem_g)
        for k in range(KB):
            k32 = jnp.int32(k)
            pltpu.make_async_copy(pos_hbm.at[idx_j.at[k32]], rows_j.at[k32],
                                  sem_g).wait()
        for k in range(KB):
            k32 = jnp.int32(k)
            pltpu.make_async_copy(pos_hbm.at[idx_i.at[k32]], rows_i.at[k32],
                                  sem_g).wait()
        for k in range(KB):
            kv = jnp.full((L,), k, jnp.int32)
            for t in range(CB // L):
                rv = iota + (t * L)
                xj = plsc.load_gather(rows_j, [kv, rv, c0])
                yj = plsc.load_gather(rows_j, [kv, rv, c1])
                zj = plsc.load_gather(rows_j, [kv, rv, c2])
                xi = plsc.load_gather(rows_i, [kv, rv, c0])
                yi = plsc.load_gather(rows_i, [kv, rv, c1])
                zi = plsc.load_gather(rows_i, [kv, rv, c2])
                dx = xj - xi
                dy = yj - yi
                dz = zj - zi
                d2 = dx * dx + dy * dy + dz * dz
                bits = lax.bitcast_convert_type(d2, jnp.int32)
                bits = 0x5F3759DF - (bits >> 1)
                y = lax.bitcast_convert_type(bits, jnp.float32)
                y = y * (1.5 - 0.5 * d2 * y * y)
                y = y * (1.5 - 0.5 * d2 * y * y)
                y = y * (1.5 - 0.5 * d2 * y * y)
                dist = jnp.where(d2 > 0.0, d2 * y, 0.0)
                k32 = jnp.int32(k)
                dist_b[k32, pl.ds(t * L, L)] = dist
                dxb[k32, pl.ds(t * L, L)] = dx
                dyb[k32, pl.ds(t * L, L)] = dy
                dzb[k32, pl.ds(t * L, L)] = dz
                ii = idx_i[k32, pl.ds(t * L, L)]
                plsc.addupdate_scatter(hist_v, [ii], ones16)
        pltpu.sync_copy(dxb, dvec_hbm.at[jnp.int32(0), sc])
        pltpu.sync_copy(dyb, dvec_hbm.at[jnp.int32(1), sc])
        pltpu.sync_copy(dzb, dvec_hbm.at[jnp.int32(2), sc])
        pltpu.sync_copy(dist_b, dist_hbm.at[sc])
        return carry

    lax.fori_loop(jnp.int32(0), n_g, _iter, 0)
    pltpu.sync_copy(hist_v, hist_hbm.at[wid])


_sc_call = functools.partial(
    pl.kernel,
    out_type=(
        jax.ShapeDtypeStruct((3, N_SC, KB, CB), jnp.float32),
        jax.ShapeDtypeStruct((N_SC, KB, CB), jnp.float32),
        jax.ShapeDtypeStruct((NW, N_NODES), jnp.int32),
    ),
    mesh=plsc.VectorSubcoreMesh(core_axis_name="c", subcore_axis_name="s",
                                num_cores=NC, num_subcores=NS),
    compiler_params=pltpu.CompilerParams(needs_layout_passes=False,
                                         use_tc_tiling_on_sc=False),
    scratch_types=[
        pltpu.MemorySpace.VMEM((KB, CB), jnp.int32),
        pltpu.MemorySpace.VMEM((KB, CB), jnp.int32),
        pltpu.MemorySpace.VMEM((KB, CB, 8), jnp.float32),
        pltpu.MemorySpace.VMEM((KB, CB, 8), jnp.float32),
        pltpu.MemorySpace.VMEM((KB, CB), jnp.float32),
        pltpu.MemorySpace.VMEM((KB, CB), jnp.float32),
        pltpu.MemorySpace.VMEM((KB, CB), jnp.float32),
        pltpu.MemorySpace.VMEM((KB, CB), jnp.float32),
        pltpu.MemorySpace.VMEM((N_NODES,), jnp.int32),
        pltpu.SemaphoreType.DMA,
    ],
)(_sc_body)


def _hist_reduce_body(parts_ref, out_ref):
    out_ref[...] = jnp.sum(parts_ref[...], axis=0, dtype=jnp.int32)


def _hist_reduce(parts):
    return pl.pallas_call(
        _hist_reduce_body,
        out_shape=jax.ShapeDtypeStruct((N_NODES,), jnp.int32),
    )(parts)


def kernel(pos, edge_index):
    ji32 = edge_index.astype(jnp.int32).reshape(2, N_SC, KB, CB)
    pos8 = jnp.pad(pos, ((0, 0), (0, 5)))
    dvec, dist, hist_parts = _sc_call(pos8, ji32)
    neighbors = _hist_reduce(hist_parts).astype(edge_index.dtype)
    zeros = jnp.zeros((N_EDGES, 3), pos.dtype)
    return (edge_index,
            dist.reshape(N_EDGES),
            dvec.reshape(3, N_EDGES).T,
            zeros,
            zeros,
            neighbors)


# speedup vs baseline: 433.4353x; 1.6104x over previous
import functools

import jax
import jax.numpy as jnp
from jax import lax
from jax.experimental import pallas as pl
from jax.experimental.pallas import tpu as pltpu
from jax.experimental.pallas import tpu_sc as plsc

N_NODES = 100000
N_EDGES = 6400000
NC = 2
NS = 16
NW = NC * NS
L = 16
CB = 128
KB = 4
CHUNK = CB * KB
N_CH = N_EDGES // CHUNK
G_FULL = N_CH // NW
G_REM = N_CH % NW


def _sc_body(pos_hbm, ji_hbm, dvec_hbm, dist_hbm, hist_hbm,
             idx_j, idx_i, rows_j, rows_i, dxb, dyb, dzb, dist_b, hist_v,
             sem_ij, sem_ii, sem_g, sem_o):
    cid = lax.axis_index("c")
    sid = lax.axis_index("s")
    wid = cid * NS + sid

    iota = lax.iota(jnp.int32, L)
    zero16 = jnp.zeros((L,), jnp.int32)
    ones16 = jnp.ones((L,), jnp.int32)
    c0 = jnp.zeros((L,), jnp.int32)
    c1 = jnp.full((L,), 1, jnp.int32)
    c2 = jnp.full((L,), 2, jnp.int32)

    def _zero(t, carry):
        hist_v[pl.ds(t * L, L)] = zero16
        return carry

    lax.fori_loop(jnp.int32(0), jnp.int32(N_NODES // L), _zero, 0)

    n_g = jnp.where(wid < G_REM, jnp.int32(G_FULL + 1), jnp.int32(G_FULL))

    def _chunk_of(g):
        return g * NW + wid

    def _idx_start(g, p):
        ch = _chunk_of(g)
        pltpu.async_copy(ji_hbm.at[jnp.int32(0), ch], idx_j.at[p], sem_ij)
        pltpu.async_copy(ji_hbm.at[jnp.int32(1), ch], idx_i.at[p], sem_ii)

    def _idx_wait(g, p):
        ch = _chunk_of(g)
        pltpu.make_async_copy(ji_hbm.at[jnp.int32(0), ch], idx_j.at[p],
                              sem_ij).wait()
        pltpu.make_async_copy(ji_hbm.at[jnp.int32(1), ch], idx_i.at[p],
                              sem_ii).wait()

    def _gather_start(p):
        for k in range(KB):
            k32 = jnp.int32(k)
            pltpu.async_copy(pos_hbm.at[idx_j.at[p, k32]],
                             rows_j.at[p, k32], sem_g)
            pltpu.async_copy(pos_hbm.at[idx_i.at[p, k32]],
                             rows_i.at[p, k32], sem_g)

    def _gather_wait(p):
        for k in range(KB):
            k32 = jnp.int32(k)
            pltpu.make_async_copy(pos_hbm.at[idx_j.at[p, k32]],
                                  rows_j.at[p, k32], sem_g).wait()
            pltpu.make_async_copy(pos_hbm.at[idx_i.at[p, k32]],
                                  rows_i.at[p, k32], sem_g).wait()

    def _out_start(g, p):
        ch = _chunk_of(g)
        pltpu.async_copy(dxb.at[p], dvec_hbm.at[jnp.int32(0), ch], sem_o)
        pltpu.async_copy(dyb.at[p], dvec_hbm.at[jnp.int32(1), ch], sem_o)
        pltpu.async_copy(dzb.at[p], dvec_hbm.at[jnp.int32(2), ch], sem_o)
        pltpu.async_copy(dist_b.at[p], dist_hbm.at[ch], sem_o)

    def _out_wait(g, p):
        ch = _chunk_of(g)
        pltpu.make_async_copy(dxb.at[p], dvec_hbm.at[jnp.int32(0), ch],
                              sem_o).wait()
        pltpu.make_async_copy(dyb.at[p], dvec_hbm.at[jnp.int32(1), ch],
                              sem_o).wait()
        pltpu.make_async_copy(dzb.at[p], dvec_hbm.at[jnp.int32(2), ch],
                              sem_o).wait()
        pltpu.make_async_copy(dist_b.at[p], dist_hbm.at[ch], sem_o).wait()

    def _compute(p):
        for k in range(KB):
            kv = jnp.full((L,), k, jnp.int32)
            k32 = jnp.int32(k)
            for t in range(CB // L):
                rv = iota + (t * L)
                xj = plsc.load_gather(rows_j.at[p], [kv, rv, c0])
                yj = plsc.load_gather(rows_j.at[p], [kv, rv, c1])
                zj = plsc.load_gather(rows_j.at[p], [kv, rv, c2])
                xi = plsc.load_gather(rows_i.at[p], [kv, rv, c0])
                yi = plsc.load_gather(rows_i.at[p], [kv, rv, c1])
                zi = plsc.load_gather(rows_i.at[p], [kv, rv, c2])
                dx = xj - xi
                dy = yj - yi
                dz = zj - zi
                d2 = dx * dx + dy * dy + dz * dz
                bits = lax.bitcast_convert_type(d2, jnp.int32)
                bits = 0x5F3759DF - (bits >> 1)
                y = lax.bitcast_convert_type(bits, jnp.float32)
                y = y * (1.5 - 0.5 * d2 * y * y)
                y = y * (1.5 - 0.5 * d2 * y * y)
                y = y * (1.5 - 0.5 * d2 * y * y)
                dist = jnp.where(d2 > 0.0, d2 * y, 0.0)
                dist_b[p, k32, pl.ds(t * L, L)] = dist
                dxb[p, k32, pl.ds(t * L, L)] = dx
                dyb[p, k32, pl.ds(t * L, L)] = dy
                dzb[p, k32, pl.ds(t * L, L)] = dz
                ii = idx_i[p, k32, pl.ds(t * L, L)]
                plsc.addupdate_scatter(hist_v, [ii], ones16)

    zero32 = jnp.int32(0)
    one32 = jnp.int32(1)
    _idx_start(zero32, zero32)
    _idx_wait(zero32, zero32)
    _gather_start(zero32)

    @pl.when(n_g >= 2)
    def _():
        _idx_start(one32, one32)

    def _iter(g, carry):
        p = lax.rem(g, jnp.int32(2))
        q = one32 - p

        _gather_wait(p)

        @pl.when(g + 1 < n_g)
        def _():
            _idx_wait(g + 1, q)
            _gather_start(q)

        @pl.when(g >= 1)
        def _():
            _out_wait(g - 1, q)

        _compute(p)
        _out_start(g, p)

        @pl.when(g + 2 < n_g)
        def _():
            _idx_start(g + 2, p)
        return carry

    lax.fori_loop(zero32, n_g, _iter, 0)

    _out_wait(n_g - 1, lax.rem(n_g - 1, jnp.int32(2)))
    pltpu.sync_copy(hist_v, hist_hbm.at[wid])


_sc_call = functools.partial(
    pl.kernel,
    out_type=(
        jax.ShapeDtypeStruct((3, N_CH, KB, CB), jnp.float32),
        jax.ShapeDtypeStruct((N_CH, KB, CB), jnp.float32),
        jax.ShapeDtypeStruct((NW, N_NODES), jnp.int32),
    ),
    mesh=plsc.VectorSubcoreMesh(core_axis_name="c", subcore_axis_name="s",
                                num_cores=NC, num_subcores=NS),
    compiler_params=pltpu.CompilerParams(needs_layout_passes=False,
                                         use_tc_tiling_on_sc=False),
    scratch_types=[
        pltpu.MemorySpace.VMEM((2, KB, CB), jnp.int32),
        pltpu.MemorySpace.VMEM((2, KB, CB), jnp.int32),
        pltpu.MemorySpace.VMEM((2, KB, CB, 8), jnp.float32),
        pltpu.MemorySpace.VMEM((2, KB, CB, 8), jnp.float32),
        pltpu.MemorySpace.VMEM((2, KB, CB), jnp.float32),
        pltpu.MemorySpace.VMEM((2, KB, CB), jnp.float32),
        pltpu.MemorySpace.VMEM((2, KB, CB), jnp.float32),
        pltpu.MemorySpace.VMEM((2, KB, CB), jnp.float32),
        pltpu.MemorySpace.VMEM((N_NODES,), jnp.int32),
        pltpu.SemaphoreType.DMA,
        pltpu.SemaphoreType.DMA,
        pltpu.SemaphoreType.DMA,
        pltpu.SemaphoreType.DMA,
    ],
)(_sc_body)


def _hist_reduce_body(parts_ref, out_ref):
    out_ref[...] = jnp.sum(parts_ref[...], axis=0, dtype=jnp.int32)


def _hist_reduce(parts):
    return pl.pallas_call(
        _hist_reduce_body,
        out_shape=jax.ShapeDtypeStruct((N_NODES,), jnp.int32),
    )(parts)


def kernel(pos, edge_index):
    ji32 = edge_index.astype(jnp.int32).reshape(2, N_CH, KB, CB)
    pos8 = jnp.pad(pos, ((0, 0), (0, 5)))
    dvec, dist, hist_parts = _sc_call(pos8, ji32)
    neighbors = _hist_reduce(hist_parts).astype(edge_index.dtype)
    zeros = jnp.zeros((N_EDGES, 3), pos.dtype)
    return (edge_index,
            dist.reshape(N_EDGES),
            dvec.reshape(3, N_EDGES).T,
            zeros,
            zeros,
            neighbors)


# speedup vs baseline: 523.8064x; 1.2085x over previous
import functools

import jax
import jax.numpy as jnp
from jax import lax
from jax.experimental import pallas as pl
from jax.experimental.pallas import tpu as pltpu
from jax.experimental.pallas import tpu_sc as plsc

N_NODES = 100000
N_EDGES = 6400000
NC = 2
NS = 16
NW = NC * NS
L = 16
CB = 128
KB = 4
CHUNK = CB * KB
N_CH = N_EDGES // CHUNK
G_FULL = N_CH // NW
G_REM = N_CH % NW


def _sc_body(pos_hbm, ji_hbm, dvec_hbm, dist_hbm, hist_hbm,
             idx_j, idx_i, rows_j, rows_i, dxb, dyb, dzb, dist_b, hist_v,
             sem_ij, sem_ii, sem_g, sem_o):
    cid = lax.axis_index("c")
    sid = lax.axis_index("s")
    wid = cid * NS + sid

    iota = lax.iota(jnp.int32, L)
    zero16 = jnp.zeros((L,), jnp.int32)
    ones16 = jnp.ones((L,), jnp.int32)
    c0 = jnp.zeros((L,), jnp.int32)
    c1 = jnp.full((L,), 1, jnp.int32)
    c2 = jnp.full((L,), 2, jnp.int32)

    def _zero(t, carry):
        hist_v[pl.ds(t * L, L)] = zero16
        return carry

    lax.fori_loop(jnp.int32(0), jnp.int32(N_NODES // L), _zero, 0)

    n_g = jnp.where(wid < G_REM, jnp.int32(G_FULL + 1), jnp.int32(G_FULL))

    def _chunk_of(g):
        return g * NW + wid

    def _idx_start(g, p):
        ch = _chunk_of(g)
        pltpu.async_copy(ji_hbm.at[jnp.int32(0), ch], idx_j.at[p], sem_ij)
        pltpu.async_copy(ji_hbm.at[jnp.int32(1), ch], idx_i.at[p], sem_ii)

    def _idx_wait(g, p):
        ch = _chunk_of(g)
        pltpu.make_async_copy(ji_hbm.at[jnp.int32(0), ch], idx_j.at[p],
                              sem_ij).wait()
        pltpu.make_async_copy(ji_hbm.at[jnp.int32(1), ch], idx_i.at[p],
                              sem_ii).wait()

    def _gather_start(p):
        for k in range(KB):
            k32 = jnp.int32(k)
            pltpu.async_copy(pos_hbm.at[idx_j.at[p, k32]],
                             rows_j.at[p, k32], sem_g)
            pltpu.async_copy(pos_hbm.at[idx_i.at[p, k32]],
                             rows_i.at[p, k32], sem_g)

    def _gather_wait(p):
        for k in range(KB):
            k32 = jnp.int32(k)
            pltpu.make_async_copy(pos_hbm.at[idx_j.at[p, k32]],
                                  rows_j.at[p, k32], sem_g).wait()
            pltpu.make_async_copy(pos_hbm.at[idx_i.at[p, k32]],
                                  rows_i.at[p, k32], sem_g).wait()

    def _out_start(g, p):
        ch = _chunk_of(g)
        pltpu.async_copy(dxb.at[p], dvec_hbm.at[jnp.int32(0), ch], sem_o)
        pltpu.async_copy(dyb.at[p], dvec_hbm.at[jnp.int32(1), ch], sem_o)
        pltpu.async_copy(dzb.at[p], dvec_hbm.at[jnp.int32(2), ch], sem_o)
        pltpu.async_copy(dist_b.at[p], dist_hbm.at[ch], sem_o)

    def _out_wait(g, p):
        ch = _chunk_of(g)
        pltpu.make_async_copy(dxb.at[p], dvec_hbm.at[jnp.int32(0), ch],
                              sem_o).wait()
        pltpu.make_async_copy(dyb.at[p], dvec_hbm.at[jnp.int32(1), ch],
                              sem_o).wait()
        pltpu.make_async_copy(dzb.at[p], dvec_hbm.at[jnp.int32(2), ch],
                              sem_o).wait()
        pltpu.make_async_copy(dist_b.at[p], dist_hbm.at[ch], sem_o).wait()

    def _compute(p):
        for k in range(KB):
            kv = jnp.full((L,), k, jnp.int32)
            k32 = jnp.int32(k)
            for t in range(CB // L):
                rv = iota + (t * L)
                xj = plsc.load_gather(rows_j.at[p], [kv, rv, c0])
                yj = plsc.load_gather(rows_j.at[p], [kv, rv, c1])
                zj = plsc.load_gather(rows_j.at[p], [kv, rv, c2])
                xi = plsc.load_gather(rows_i.at[p], [kv, rv, c0])
                yi = plsc.load_gather(rows_i.at[p], [kv, rv, c1])
                zi = plsc.load_gather(rows_i.at[p], [kv, rv, c2])
                dx = xj - xi
                dy = yj - yi
                dz = zj - zi
                d2 = dx * dx + dy * dy + dz * dz
                bits = lax.bitcast_convert_type(d2, jnp.int32)
                bits = 0x5F3759DF - (bits >> 1)
                y = lax.bitcast_convert_type(bits, jnp.float32)
                y = y * (1.5 - 0.5 * d2 * y * y)
                y = y * (1.5 - 0.5 * d2 * y * y)
                y = y * (1.5 - 0.5 * d2 * y * y)
                dist = jnp.where(d2 > 0.0, d2 * y, 0.0)
                dist_b[p, k32, pl.ds(t * L, L)] = dist
                dxb[p, k32, pl.ds(t * L, L)] = dx
                dyb[p, k32, pl.ds(t * L, L)] = dy
                dzb[p, k32, pl.ds(t * L, L)] = dz
                ii = idx_i[p, k32, pl.ds(t * L, L)]
                plsc.addupdate_scatter(hist_v, [ii], ones16)

    zero32 = jnp.int32(0)
    one32 = jnp.int32(1)
    _idx_start(zero32, zero32)
    _idx_wait(zero32, zero32)
    _gather_start(zero32)

    @pl.when(n_g >= 2)
    def _():
        _idx_start(one32, one32)

    def _iter(g, carry):
        p = lax.rem(g, jnp.int32(2))
        q = one32 - p

        _gather_wait(p)

        @pl.when(g + 1 < n_g)
        def _():
            _idx_wait(g + 1, q)
            _gather_start(q)

        @pl.when(g >= 1)
        def _():
            _out_wait(g - 1, q)

        _compute(p)
        _out_start(g, p)

        @pl.when(g + 2 < n_g)
        def _():
            _idx_start(g + 2, p)
        return carry

    lax.fori_loop(zero32, n_g, _iter, 0)

    _out_wait(n_g - 1, lax.rem(n_g - 1, jnp.int32(2)))
    pltpu.sync_copy(hist_v, hist_hbm.at[wid])


_sc_call = functools.partial(
    pl.kernel,
    out_type=(
        jax.ShapeDtypeStruct((3, N_CH, KB, CB), jnp.float32),
        jax.ShapeDtypeStruct((N_CH, KB, CB), jnp.float32),
        jax.ShapeDtypeStruct((NW, N_NODES), jnp.int32),
    ),
    mesh=plsc.VectorSubcoreMesh(core_axis_name="c", subcore_axis_name="s",
                                num_cores=NC, num_subcores=NS),
    compiler_params=pltpu.CompilerParams(needs_layout_passes=False,
                                         use_tc_tiling_on_sc=False),
    scratch_types=[
        pltpu.MemorySpace.VMEM((2, KB, CB), jnp.int32),
        pltpu.MemorySpace.VMEM((2, KB, CB), jnp.int32),
        pltpu.MemorySpace.VMEM((2, KB, CB, 8), jnp.float32),
        pltpu.MemorySpace.VMEM((2, KB, CB, 8), jnp.float32),
        pltpu.MemorySpace.VMEM((2, KB, CB), jnp.float32),
        pltpu.MemorySpace.VMEM((2, KB, CB), jnp.float32),
        pltpu.MemorySpace.VMEM((2, KB, CB), jnp.float32),
        pltpu.MemorySpace.VMEM((2, KB, CB), jnp.float32),
        pltpu.MemorySpace.VMEM((N_NODES,), jnp.int32),
        pltpu.SemaphoreType.DMA,
        pltpu.SemaphoreType.DMA,
        pltpu.SemaphoreType.DMA,
        pltpu.SemaphoreType.DMA,
    ],
)(_sc_body)


def _hist_reduce_body(parts_ref, out_ref):
    out_ref[...] = jnp.sum(parts_ref[...], axis=0, dtype=jnp.int32)


def _hist_reduce(parts):
    return pl.pallas_call(
        _hist_reduce_body,
        out_shape=jax.ShapeDtypeStruct((N_NODES,), jnp.int32),
    )(parts)


def kernel(pos, edge_index):
    ji32 = edge_index.astype(jnp.int32).reshape(2, N_CH, KB, CB)
    pos8 = jnp.pad(pos, ((0, 0), (0, 5)))
    dvec, dist, hist_parts = _sc_call(pos8, ji32)
    neighbors = _hist_reduce(hist_parts).astype(edge_index.dtype)
    zeros = jnp.zeros((N_EDGES, 3), pos.dtype)
    ei_out = ji32.reshape(2, N_EDGES).astype(edge_index.dtype)
    return (ei_out,
            dist.reshape(N_EDGES),
            dvec.reshape(3, N_EDGES).T,
            zeros,
            zeros,
            neighbors)


# speedup vs baseline: 540.4283x; 1.0317x over previous
import functools

import jax
import jax.numpy as jnp
from jax import lax
from jax.experimental import pallas as pl
from jax.experimental.pallas import tpu as pltpu
from jax.experimental.pallas import tpu_sc as plsc

N_NODES = 100000
N_EDGES = 6400000
NC = 2
NS = 16
NW = NC * NS
L = 16
CB = 128
KB = 4
CHUNK = CB * KB
N_CH = N_EDGES // CHUNK
G_FULL = N_CH // NW
G_REM = N_CH % NW


def _sc_body(pos_hbm, ji_hbm, dvec_hbm, dist_hbm, hist_hbm,
             idx_j, idx_i, rows_j, rows_i, vb, dist_b, hist_v,
             sem_ij, sem_ii, sem_g, sem_o):
    cid = lax.axis_index("c")
    sid = lax.axis_index("s")
    wid = cid * NS + sid

    iota = lax.iota(jnp.int32, L)
    zero16 = jnp.zeros((L,), jnp.int32)
    ones16 = jnp.ones((L,), jnp.int32)
    c0 = jnp.zeros((L,), jnp.int32)
    c1 = jnp.full((L,), 1, jnp.int32)
    c2 = jnp.full((L,), 2, jnp.int32)

    def _zero(t, carry):
        hist_v[pl.ds(t * L, L)] = zero16
        return carry

    lax.fori_loop(jnp.int32(0), jnp.int32(N_NODES // L), _zero, 0)

    zerof = jnp.zeros((L,), jnp.float32)
    for p in range(2):
        for k in range(KB):
            for t in range(CB // L):
                vb[jnp.int32(p), jnp.int32(k), jnp.int32(3),
                   pl.ds(t * L, L)] = zerof

    n_g = jnp.where(wid < G_REM, jnp.int32(G_FULL + 1), jnp.int32(G_FULL))

    def _chunk_of(g):
        return g * NW + wid

    def _idx_start(g, p):
        ch = _chunk_of(g)
        pltpu.async_copy(ji_hbm.at[jnp.int32(0), ch], idx_j.at[p], sem_ij)
        pltpu.async_copy(ji_hbm.at[jnp.int32(1), ch], idx_i.at[p], sem_ii)

    def _idx_wait(g, p):
        ch = _chunk_of(g)
        pltpu.make_async_copy(ji_hbm.at[jnp.int32(0), ch], idx_j.at[p],
                              sem_ij).wait()
        pltpu.make_async_copy(ji_hbm.at[jnp.int32(1), ch], idx_i.at[p],
                              sem_ii).wait()

    def _gather_start(p):
        for k in range(KB):
            k32 = jnp.int32(k)
            pltpu.async_copy(pos_hbm.at[idx_j.at[p, k32]],
                             rows_j.at[p, k32], sem_g)
            pltpu.async_copy(pos_hbm.at[idx_i.at[p, k32]],
                             rows_i.at[p, k32], sem_g)

    def _gather_wait(p):
        for k in range(KB):
            k32 = jnp.int32(k)
            pltpu.make_async_copy(pos_hbm.at[idx_j.at[p, k32]],
                                  rows_j.at[p, k32], sem_g).wait()
            pltpu.make_async_copy(pos_hbm.at[idx_i.at[p, k32]],
                                  rows_i.at[p, k32], sem_g).wait()

    def _out_start(g, p):
        ch = _chunk_of(g)
        pltpu.async_copy(vb.at[p], dvec_hbm.at[ch], sem_o)
        pltpu.async_copy(dist_b.at[p], dist_hbm.at[ch], sem_o)

    def _out_wait(g, p):
        ch = _chunk_of(g)
        pltpu.make_async_copy(vb.at[p], dvec_hbm.at[ch], sem_o).wait()
        pltpu.make_async_copy(dist_b.at[p], dist_hbm.at[ch], sem_o).wait()

    def _compute(p):
        c0i = jnp.int32(0)
        c1i = jnp.int32(1)
        c2i = jnp.int32(2)
        for k in range(KB):
            kv = jnp.full((L,), k, jnp.int32)
            k32 = jnp.int32(k)
            for t in range(CB // L):
                rv = iota + (t * L)
                xj = plsc.load_gather(rows_j.at[p], [kv, rv, c0])
                yj = plsc.load_gather(rows_j.at[p], [kv, rv, c1])
                zj = plsc.load_gather(rows_j.at[p], [kv, rv, c2])
                xi = plsc.load_gather(rows_i.at[p], [kv, rv, c0])
                yi = plsc.load_gather(rows_i.at[p], [kv, rv, c1])
                zi = plsc.load_gather(rows_i.at[p], [kv, rv, c2])
                dx = xj - xi
                dy = yj - yi
                dz = zj - zi
                d2 = dx * dx + dy * dy + dz * dz
                bits = lax.bitcast_convert_type(d2, jnp.int32)
                bits = 0x5F3759DF - (bits >> 1)
                y = lax.bitcast_convert_type(bits, jnp.float32)
                y = y * (1.5 - 0.5 * d2 * y * y)
                y = y * (1.5 - 0.5 * d2 * y * y)
                y = y * (1.5 - 0.5 * d2 * y * y)
                dist = jnp.where(d2 > 0.0, d2 * y, 0.0)
                dist_b[p, k32, pl.ds(t * L, L)] = dist
                vb[p, k32, c0i, pl.ds(t * L, L)] = dx
                vb[p, k32, c1i, pl.ds(t * L, L)] = dy
                vb[p, k32, c2i, pl.ds(t * L, L)] = dz
                ii = idx_i[p, k32, pl.ds(t * L, L)]
                plsc.addupdate_scatter(hist_v, [ii], ones16)

    zero32 = jnp.int32(0)
    one32 = jnp.int32(1)
    _idx_start(zero32, zero32)
    _idx_wait(zero32, zero32)
    _gather_start(zero32)

    @pl.when(n_g >= 2)
    def _():
        _idx_start(one32, one32)

    def _iter(g, carry):
        p = lax.rem(g, jnp.int32(2))
        q = one32 - p

        _gather_wait(p)

        @pl.when(g + 1 < n_g)
        def _():
            _idx_wait(g + 1, q)
            _gather_start(q)

        @pl.when(g >= 1)
        def _():
            _out_wait(g - 1, q)

        _compute(p)
        _out_start(g, p)

        @pl.when(g + 2 < n_g)
        def _():
            _idx_start(g + 2, p)
        return carry

    lax.fori_loop(zero32, n_g, _iter, 0)

    _out_wait(n_g - 1, lax.rem(n_g - 1, jnp.int32(2)))
    pltpu.sync_copy(hist_v, hist_hbm.at[wid])


_sc_call = functools.partial(
    pl.kernel,
    out_type=(
        jax.ShapeDtypeStruct((N_CH, KB, 4, CB), jnp.float32),
        jax.ShapeDtypeStruct((N_CH, KB, CB), jnp.float32),
        jax.ShapeDtypeStruct((NW, N_NODES), jnp.int32),
    ),
    mesh=plsc.VectorSubcoreMesh(core_axis_name="c", subcore_axis_name="s",
                                num_cores=NC, num_subcores=NS),
    compiler_params=pltpu.CompilerParams(needs_layout_passes=False,
                                         use_tc_tiling_on_sc=False),
    scratch_types=[
        pltpu.MemorySpace.VMEM((2, KB, CB), jnp.int32),
        pltpu.MemorySpace.VMEM((2, KB, CB), jnp.int32),
        pltpu.MemorySpace.VMEM((2, KB, CB, 8), jnp.float32),
        pltpu.MemorySpace.VMEM((2, KB, CB, 8), jnp.float32),
        pltpu.MemorySpace.VMEM((2, KB, 4, CB), jnp.float32),
        pltpu.MemorySpace.VMEM((2, KB, CB), jnp.float32),
        pltpu.MemorySpace.VMEM((N_NODES,), jnp.int32),
        pltpu.SemaphoreType.DMA,
        pltpu.SemaphoreType.DMA,
        pltpu.SemaphoreType.DMA,
        pltpu.SemaphoreType.DMA,
    ],
)(_sc_body)


def _hist_reduce_body(parts_ref, out_ref):
    out_ref[...] = jnp.sum(parts_ref[...], axis=0, dtype=jnp.int32)


def _hist_reduce(parts):
    return pl.pallas_call(
        _hist_reduce_body,
        out_shape=jax.ShapeDtypeStruct((N_NODES,), jnp.int32),
    )(parts)


def kernel(pos, edge_index):
    ji32 = edge_index.astype(jnp.int32).reshape(2, N_CH, KB, CB)
    pos8 = jnp.pad(pos, ((0, 0), (0, 5)))
    dvec, dist, hist_parts = _sc_call(pos8, ji32)
    neighbors = _hist_reduce(hist_parts).astype(edge_index.dtype)
    zeros = jnp.zeros((N_EDGES, 3), pos.dtype)
    ei_out = ji32.reshape(2, N_EDGES).astype(edge_index.dtype)
    dv4 = dvec.reshape(N_EDGES // CB, 4, CB)
    return (ei_out,
            dist.reshape(N_EDGES),
            dv4[:, :3, :].transpose(0, 2, 1).reshape(N_EDGES, 3),
            zeros,
            zeros,
            neighbors)


# speedup vs baseline: 542.1960x; 1.0033x over previous
import functools

import jax
import jax.numpy as jnp
from jax import lax
from jax.experimental import pallas as pl
from jax.experimental.pallas import tpu as pltpu
from jax.experimental.pallas import tpu_sc as plsc

N_NODES = 100000
N_EDGES = 6400000
NC = 2
NS = 16
NW = NC * NS
L = 16
CB = 128
KB = 4
CHUNK = CB * KB
N_CH = N_EDGES // CHUNK
G_FULL = N_CH // NW
G_REM = N_CH % NW


def _sc_body(pos_hbm, ji_hbm, dvec_hbm, dist_hbm, hist_hbm,
             idx_j, idx_i, rows_j, rows_i, vb, dist_b, hist_v,
             sem_ij, sem_ii, sem_g, sem_o):
    cid = lax.axis_index("c")
    sid = lax.axis_index("s")
    wid = cid * NS + sid

    iota = lax.iota(jnp.int32, L)
    zero16 = jnp.zeros((L,), jnp.int32)
    ones16 = jnp.ones((L,), jnp.int32)
    c0 = jnp.zeros((L,), jnp.int32)
    c1 = jnp.full((L,), 1, jnp.int32)
    c2 = jnp.full((L,), 2, jnp.int32)

    def _zero(t, carry):
        hist_v[pl.ds(t * L, L)] = zero16
        return carry

    lax.fori_loop(jnp.int32(0), jnp.int32(N_NODES // L), _zero, 0)

    zerof = jnp.zeros((L,), jnp.float32)
    for p in range(2):
        for k in range(KB):
            for t in range(CB // L):
                vb[jnp.int32(p), jnp.int32(k), jnp.int32(3),
                   pl.ds(t * L, L)] = zerof

    n_g = jnp.where(wid < G_REM, jnp.int32(G_FULL + 1), jnp.int32(G_FULL))

    def _chunk_of(g):
        return g * NW + wid

    def _idx_start(g, p):
        e0 = _chunk_of(g) * CHUNK
        pltpu.async_copy(ji_hbm.at[jnp.int32(0), pl.ds(e0, CHUNK)],
                         idx_j.at[p], sem_ij)
        pltpu.async_copy(ji_hbm.at[jnp.int32(1), pl.ds(e0, CHUNK)],
                         idx_i.at[p], sem_ii)

    def _idx_wait(g, p):
        e0 = _chunk_of(g) * CHUNK
        pltpu.make_async_copy(ji_hbm.at[jnp.int32(0), pl.ds(e0, CHUNK)],
                              idx_j.at[p], sem_ij).wait()
        pltpu.make_async_copy(ji_hbm.at[jnp.int32(1), pl.ds(e0, CHUNK)],
                              idx_i.at[p], sem_ii).wait()

    def _gather_start(p):
        for k in range(KB):
            k32 = jnp.int32(k)
            pltpu.async_copy(pos_hbm.at[idx_j.at[p, pl.ds(k * CB, CB)]],
                             rows_j.at[p, k32], sem_g)
            pltpu.async_copy(pos_hbm.at[idx_i.at[p, pl.ds(k * CB, CB)]],
                             rows_i.at[p, k32], sem_g)

    def _gather_wait(p):
        for k in range(KB):
            k32 = jnp.int32(k)
            pltpu.make_async_copy(pos_hbm.at[idx_j.at[p, pl.ds(k * CB, CB)]],
                                  rows_j.at[p, k32], sem_g).wait()
            pltpu.make_async_copy(pos_hbm.at[idx_i.at[p, pl.ds(k * CB, CB)]],
                                  rows_i.at[p, k32], sem_g).wait()

    def _out_start(g, p):
        ch = _chunk_of(g)
        pltpu.async_copy(vb.at[p], dvec_hbm.at[ch], sem_o)
        pltpu.async_copy(dist_b.at[p], dist_hbm.at[ch], sem_o)

    def _out_wait(g, p):
        ch = _chunk_of(g)
        pltpu.make_async_copy(vb.at[p], dvec_hbm.at[ch], sem_o).wait()
        pltpu.make_async_copy(dist_b.at[p], dist_hbm.at[ch], sem_o).wait()

    def _compute(p):
        c0i = jnp.int32(0)
        c1i = jnp.int32(1)
        c2i = jnp.int32(2)
        for k in range(KB):
            kv = jnp.full((L,), k, jnp.int32)
            k32 = jnp.int32(k)
            for t in range(CB // L):
                rv = iota + (t * L)
                xj = plsc.load_gather(rows_j.at[p], [kv, rv, c0])
                yj = plsc.load_gather(rows_j.at[p], [kv, rv, c1])
                zj = plsc.load_gather(rows_j.at[p], [kv, rv, c2])
                xi = plsc.load_gather(rows_i.at[p], [kv, rv, c0])
                yi = plsc.load_gather(rows_i.at[p], [kv, rv, c1])
                zi = plsc.load_gather(rows_i.at[p], [kv, rv, c2])
                dx = xj - xi
                dy = yj - yi
                dz = zj - zi
                d2 = dx * dx + dy * dy + dz * dz
                bits = lax.bitcast_convert_type(d2, jnp.int32)
                bits = 0x5F3759DF - (bits >> 1)
                y = lax.bitcast_convert_type(bits, jnp.float32)
                y = y * (1.5 - 0.5 * d2 * y * y)
                y = y * (1.5 - 0.5 * d2 * y * y)
                y = y * (1.5 - 0.5 * d2 * y * y)
                dist = jnp.where(d2 > 0.0, d2 * y, 0.0)
                dist_b[p, k32, pl.ds(t * L, L)] = dist
                vb[p, k32, c0i, pl.ds(t * L, L)] = dx
                vb[p, k32, c1i, pl.ds(t * L, L)] = dy
                vb[p, k32, c2i, pl.ds(t * L, L)] = dz
                ii = idx_i[p, pl.ds(k * CB + t * L, L)]
                plsc.addupdate_scatter(hist_v, [ii], ones16)

    zero32 = jnp.int32(0)
    one32 = jnp.int32(1)
    _idx_start(zero32, zero32)
    _idx_wait(zero32, zero32)
    _gather_start(zero32)

    @pl.when(n_g >= 2)
    def _():
        _idx_start(one32, one32)

    def _iter(g, carry):
        p = lax.rem(g, jnp.int32(2))
        q = one32 - p

        _gather_wait(p)

        @pl.when(g + 1 < n_g)
        def _():
            _idx_wait(g + 1, q)
            _gather_start(q)

        @pl.when(g >= 1)
        def _():
            _out_wait(g - 1, q)

        _compute(p)
        _out_start(g, p)

        @pl.when(g + 2 < n_g)
        def _():
            _idx_start(g + 2, p)
        return carry

    lax.fori_loop(zero32, n_g, _iter, 0)

    _out_wait(n_g - 1, lax.rem(n_g - 1, jnp.int32(2)))
    pltpu.sync_copy(hist_v, hist_hbm.at[wid])


_sc_call = functools.partial(
    pl.kernel,
    out_type=(
        jax.ShapeDtypeStruct((N_CH, KB, 4, CB), jnp.float32),
        jax.ShapeDtypeStruct((N_CH, KB, CB), jnp.float32),
        jax.ShapeDtypeStruct((NW, N_NODES), jnp.int32),
    ),
    mesh=plsc.VectorSubcoreMesh(core_axis_name="c", subcore_axis_name="s",
                                num_cores=NC, num_subcores=NS),
    compiler_params=pltpu.CompilerParams(needs_layout_passes=False,
                                         use_tc_tiling_on_sc=False),
    scratch_types=[
        pltpu.MemorySpace.VMEM((2, CHUNK), jnp.int32),
        pltpu.MemorySpace.VMEM((2, CHUNK), jnp.int32),
        pltpu.MemorySpace.VMEM((2, KB, CB, 8), jnp.float32),
        pltpu.MemorySpace.VMEM((2, KB, CB, 8), jnp.float32),
        pltpu.MemorySpace.VMEM((2, KB, 4, CB), jnp.float32),
        pltpu.MemorySpace.VMEM((2, KB, CB), jnp.float32),
        pltpu.MemorySpace.VMEM((N_NODES,), jnp.int32),
        pltpu.SemaphoreType.DMA,
        pltpu.SemaphoreType.DMA,
        pltpu.SemaphoreType.DMA,
        pltpu.SemaphoreType.DMA,
    ],
)(_sc_body)


def _hist_reduce_body(parts_ref, out_ref):
    out_ref[...] = jnp.sum(parts_ref[...], axis=0, dtype=jnp.int32)


def _hist_reduce(parts):
    return pl.pallas_call(
        _hist_reduce_body,
        out_shape=jax.ShapeDtypeStruct((N_NODES,), jnp.int32),
    )(parts)


def kernel(pos, edge_index):
    ji32 = lax.bitcast_convert_type(edge_index.astype(jnp.uint32), jnp.int32)
    pos8 = jnp.pad(pos, ((0, 0), (0, 5)))
    dvec, dist, hist_parts = _sc_call(pos8, ji32)
    neighbors = _hist_reduce(hist_parts).astype(edge_index.dtype)
    zeros = jnp.zeros((N_EDGES, 3), pos.dtype)
    ei_out = ji32.astype(edge_index.dtype)
    dv4 = dvec.reshape(N_EDGES // CB, 4, CB)
    return (ei_out,
            dist.reshape(N_EDGES),
            dv4[:, :3, :].transpose(0, 2, 1).reshape(N_EDGES, 3),
            zeros,
            zeros,
            neighbors)


# speedup vs baseline: 547.9464x; 1.0106x over previous
import functools

import jax
import jax.numpy as jnp
from jax import lax
from jax.experimental import pallas as pl
from jax.experimental.pallas import tpu as pltpu
from jax.experimental.pallas import tpu_sc as plsc

N_NODES = 100000
N_EDGES = 6400000
NC = 2
NS = 16
NW = NC * NS
L = 16
CB = 128
KB = 4
CHUNK = CB * KB
N_CH = N_EDGES // CHUNK
G_FULL = N_CH // NW
G_REM = N_CH % NW


def _sc_body(pos_hbm, ji_hbm, dvec_hbm, dist_hbm, hist_hbm,
             idx_j, idx_i, rows_j, rows_i, vb, dist_b, hist_v,
             sem_ij, sem_ii, sem_g, sem_o):
    cid = lax.axis_index("c")
    sid = lax.axis_index("s")
    wid = cid * NS + sid

    iota = lax.iota(jnp.int32, L)
    zero16 = jnp.zeros((L,), jnp.int32)
    ones16 = jnp.ones((L,), jnp.int32)
    c0 = jnp.zeros((L,), jnp.int32)
    c1 = jnp.full((L,), 1, jnp.int32)
    c2 = jnp.full((L,), 2, jnp.int32)

    def _zero(t, carry):
        hist_v[pl.ds(t * L, L)] = zero16
        return carry

    lax.fori_loop(jnp.int32(0), jnp.int32(N_NODES // L), _zero, 0)

    zerof = jnp.zeros((L,), jnp.float32)
    for p in range(2):
        for k in range(KB):
            for t in range(CB // L):
                vb[jnp.int32(p), jnp.int32(k), jnp.int32(3),
                   pl.ds(t * L, L)] = zerof

    n_g = jnp.where(wid < G_REM, jnp.int32(G_FULL + 1), jnp.int32(G_FULL))

    def _chunk_of(g):
        return g * NW + wid

    def _idx_start(g, p):
        e0 = _chunk_of(g) * CHUNK
        pltpu.async_copy(ji_hbm.at[jnp.int32(0), pl.ds(e0, CHUNK)],
                         idx_j.at[p], sem_ij)
        pltpu.async_copy(ji_hbm.at[jnp.int32(1), pl.ds(e0, CHUNK)],
                         idx_i.at[p], sem_ii)

    def _idx_wait(g, p):
        e0 = _chunk_of(g) * CHUNK
        pltpu.make_async_copy(ji_hbm.at[jnp.int32(0), pl.ds(e0, CHUNK)],
                              idx_j.at[p], sem_ij).wait()
        pltpu.make_async_copy(ji_hbm.at[jnp.int32(1), pl.ds(e0, CHUNK)],
                              idx_i.at[p], sem_ii).wait()

    def _gather_start(p):
        for k in range(KB):
            k32 = jnp.int32(k)
            pltpu.async_copy(pos_hbm.at[idx_j.at[p, pl.ds(k * CB, CB)]],
                             rows_j.at[p, k32], sem_g)
            pltpu.async_copy(pos_hbm.at[idx_i.at[p, pl.ds(k * CB, CB)]],
                             rows_i.at[p, k32], sem_g)

    def _gather_wait(p):
        for k in range(KB):
            k32 = jnp.int32(k)
            pltpu.make_async_copy(pos_hbm.at[idx_j.at[p, pl.ds(k * CB, CB)]],
                                  rows_j.at[p, k32], sem_g).wait()
            pltpu.make_async_copy(pos_hbm.at[idx_i.at[p, pl.ds(k * CB, CB)]],
                                  rows_i.at[p, k32], sem_g).wait()

    def _out_start(g, p):
        ch = _chunk_of(g)
        pltpu.async_copy(vb.at[p], dvec_hbm.at[ch], sem_o)
        pltpu.async_copy(dist_b.at[p], dist_hbm.at[ch], sem_o)

    def _out_wait(g, p):
        ch = _chunk_of(g)
        pltpu.make_async_copy(vb.at[p], dvec_hbm.at[ch], sem_o).wait()
        pltpu.make_async_copy(dist_b.at[p], dist_hbm.at[ch], sem_o).wait()

    def _compute(p):
        c0i = jnp.int32(0)
        c1i = jnp.int32(1)
        c2i = jnp.int32(2)
        for k in range(KB):
            kv = jnp.full((L,), k, jnp.int32)
            k32 = jnp.int32(k)
            for t in range(CB // L):
                rv = iota + (t * L)
                xj = plsc.load_gather(rows_j.at[p], [kv, rv, c0])
                yj = plsc.load_gather(rows_j.at[p], [kv, rv, c1])
                zj = plsc.load_gather(rows_j.at[p], [kv, rv, c2])
                xi = plsc.load_gather(rows_i.at[p], [kv, rv, c0])
                yi = plsc.load_gather(rows_i.at[p], [kv, rv, c1])
                zi = plsc.load_gather(rows_i.at[p], [kv, rv, c2])
                dx = xj - xi
                dy = yj - yi
                dz = zj - zi
                d2 = dx * dx + dy * dy + dz * dz
                bits = lax.bitcast_convert_type(d2, jnp.int32)
                bits = 0x5F3759DF - (bits >> 1)
                y = lax.bitcast_convert_type(bits, jnp.float32)
                y = y * (1.5 - 0.5 * d2 * y * y)
                y = y * (1.5 - 0.5 * d2 * y * y)
                y = y * (1.5 - 0.5 * d2 * y * y)
                dist = jnp.where(d2 > 0.0, d2 * y, 0.0)
                dist_b[p, k32, pl.ds(t * L, L)] = dist
                vb[p, k32, c0i, pl.ds(t * L, L)] = dx
                vb[p, k32, c1i, pl.ds(t * L, L)] = dy
                vb[p, k32, c2i, pl.ds(t * L, L)] = dz
                ii = idx_i[p, pl.ds(k * CB + t * L, L)]
                plsc.addupdate_scatter(hist_v, [ii], ones16)

    zero32 = jnp.int32(0)
    one32 = jnp.int32(1)
    _idx_start(zero32, zero32)
    _idx_wait(zero32, zero32)
    _gather_start(zero32)

    @pl.when(n_g >= 2)
    def _():
        _idx_start(one32, one32)

    def _iter(g, carry):
        p = lax.rem(g, jnp.int32(2))
        q = one32 - p

        _gather_wait(p)

        @pl.when(g + 1 < n_g)
        def _():
            _idx_wait(g + 1, q)
            _gather_start(q)

        @pl.when(g >= 1)
        def _():
            _out_wait(g - 1, q)

        _compute(p)
        _out_start(g, p)

        @pl.when(g + 2 < n_g)
        def _():
            _idx_start(g + 2, p)
        return carry

    lax.fori_loop(zero32, n_g, _iter, 0)

    _out_wait(n_g - 1, lax.rem(n_g - 1, jnp.int32(2)))
    pltpu.sync_copy(hist_v, hist_hbm.at[wid])


_sc_call = functools.partial(
    pl.kernel,
    out_type=(
        jax.ShapeDtypeStruct((N_CH, KB, 4, CB), jnp.float32),
        jax.ShapeDtypeStruct((N_CH, KB, CB), jnp.float32),
        jax.ShapeDtypeStruct((NW, N_NODES), jnp.int32),
    ),
    mesh=plsc.VectorSubcoreMesh(core_axis_name="c", subcore_axis_name="s",
                                num_cores=NC, num_subcores=NS),
    compiler_params=pltpu.CompilerParams(needs_layout_passes=False,
                                         use_tc_tiling_on_sc=False),
    scratch_types=[
        pltpu.MemorySpace.VMEM((2, CHUNK), jnp.int32),
        pltpu.MemorySpace.VMEM((2, CHUNK), jnp.int32),
        pltpu.MemorySpace.VMEM((2, KB, CB, 8), jnp.float32),
        pltpu.MemorySpace.VMEM((2, KB, CB, 8), jnp.float32),
        pltpu.MemorySpace.VMEM((2, KB, 4, CB), jnp.float32),
        pltpu.MemorySpace.VMEM((2, KB, CB), jnp.float32),
        pltpu.MemorySpace.VMEM((N_NODES,), jnp.int32),
        pltpu.SemaphoreType.DMA,
        pltpu.SemaphoreType.DMA,
        pltpu.SemaphoreType.DMA,
        pltpu.SemaphoreType.DMA,
    ],
)(_sc_body)


def _hist_reduce_body(parts_ref, out_ref):
    out_ref[...] = jnp.sum(parts_ref[...], axis=0, dtype=jnp.int32)


def _hist_reduce(parts):
    return pl.pallas_call(
        _hist_reduce_body,
        out_shape=jax.ShapeDtypeStruct((N_NODES,), jnp.int32),
    )(parts)


def kernel(pos, edge_index):
    ji32 = lax.bitcast_convert_type(edge_index.astype(jnp.uint32), jnp.int32)
    pos8 = jnp.pad(pos, ((0, 0), (0, 5)))
    dvec, dist, hist_parts = _sc_call(pos8, ji32)
    neighbors = _hist_reduce(hist_parts).astype(edge_index.dtype)
    zeros = jnp.zeros((N_EDGES, 3), pos.dtype)
    ei_out = lax.bitcast_convert_type(ji32, jnp.uint32).astype(edge_index.dtype)
    dv4 = dvec.reshape(N_EDGES // CB, 4, CB)
    return (ei_out,
            dist.reshape(N_EDGES),
            dv4[:, :3, :].transpose(0, 2, 1).reshape(N_EDGES, 3),
            zeros,
            zeros,
            neighbors)
